# baseline (speedup 1.0000x reference)
import functools

import jax
import jax.numpy as jnp
from jax import lax
from jax.experimental import pallas as pl
from jax.experimental.pallas import tpu as pltpu
from jax.experimental.pallas import tpu_sc as plsc

NUM_EMBEDDINGS = 1000000
DIM = 32
TOT = 16384 * 100
NC = 2
NS = 16
NW = NC * NS
PER_W = TOT // NW
CHUNK = 128
NCHUNK = PER_W // CHUNK
NBUF = 4
DEPTH = 2
PADW = 128
NG = NCHUNK // NBUF

_mesh = plsc.VectorSubcoreMesh(core_axis_name="c", subcore_axis_name="s")


@functools.partial(
    pl.kernel,
    mesh=_mesh,
    out_type=jax.ShapeDtypeStruct((TOT, DIM), jnp.float32),
    compiler_params=pltpu.CompilerParams(use_tc_tiling_on_sc=False),
    scratch_types=[
        pltpu.VMEM((NCHUNK, CHUNK), jnp.int32),
        pltpu.VMEM((NBUF, CHUNK, PADW), jnp.float32),
        pltpu.SemaphoreType.DMA,
        pltpu.SemaphoreType.DMA,
    ],
)
def _emb_lookup(idx_hbm, table_hbm, out_hbm, idx_v, rows_v, gsem, osem):
    wid = lax.axis_index("s") * NC + lax.axis_index("c")
    base = wid * PER_W

    pltpu.sync_copy(idx_hbm.at[wid], idx_v)

    def start_gather(j, slot):
        pltpu.async_copy(table_hbm.at[idx_v.at[j]], rows_v.at[slot], gsem)

    def wait_gather(j, slot):
        pltpu.make_async_copy(
            table_hbm.at[idx_v.at[j]], rows_v.at[slot], gsem
        ).wait()

    def start_store(j, slot):
        pltpu.async_copy(
            rows_v.at[slot, slice(None), pl.ds(0, DIM)],
            out_hbm.at[pl.ds(base + j * CHUNK, CHUNK)],
            osem,
        )

    def wait_store(j, slot):
        pltpu.make_async_copy(
            rows_v.at[slot, slice(None), pl.ds(0, DIM)],
            out_hbm.at[pl.ds(base + j * CHUNK, CHUNK)],
            osem,
        ).wait()

    for b in range(NBUF):
        start_gather(b, b)
        if b >= DEPTH:
            wait_gather(b - DEPTH, b - DEPTH)
            start_store(b - DEPTH, b - DEPTH)

    def round_body(g, carry):
        for b in range(NBUF):
            j = g * NBUF + b
            wait_store(j - NBUF, b)
            start_gather(j, b)
            jd = j - DEPTH
            sd = (b - DEPTH) % NBUF
            wait_gather(jd, sd)
            start_store(jd, sd)
        return carry

    lax.fori_loop(1, NG, round_body, 0)

    for b in range(NBUF):
        j = NG * NBUF + b
        wait_store(j - NBUF, b)
        if b < DEPTH:
            jd = j - DEPTH
            sd = (b - DEPTH) % NBUF
            wait_gather(jd, sd)
            start_store(jd, sd)


def kernel(input_, weight):
    idx = input_.astype(jnp.int32).reshape(NW, NCHUNK, CHUNK)
    wpad = jax.lax.pad(
        weight, jnp.float32(0), ((0, 0, 0), (0, PADW - DIM, 0))
    )
    out = _emb_lookup(idx, wpad)
    return out.reshape(input_.shape[0], input_.shape[1], DIM)


# speedup vs baseline: 1.0510x; 1.0510x over previous
import functools

import jax
import jax.numpy as jnp
from jax import lax
from jax.experimental import pallas as pl
from jax.experimental.pallas import tpu as pltpu
from jax.experimental.pallas import tpu_sc as plsc

NUM_EMBEDDINGS = 1000000
DIM = 32
TOT = 16384 * 100
NC = 2
NS = 16
NW = NC * NS
PER_W = TOT // NW
CHUNK = 128
NCHUNK = PER_W // CHUNK
NBUF = 4
DEPTH = 2
PADW = 128
NG = NCHUNK // NBUF

_mesh = plsc.VectorSubcoreMesh(core_axis_name="c", subcore_axis_name="s")


@functools.partial(
    pl.kernel,
    mesh=_mesh,
    out_type=jax.ShapeDtypeStruct((TOT, PADW), jnp.float32),
    compiler_params=pltpu.CompilerParams(use_tc_tiling_on_sc=True),
    scratch_types=[
        pltpu.VMEM((NCHUNK, CHUNK), jnp.int32),
        pltpu.VMEM((NBUF, CHUNK, PADW), jnp.float32),
        pltpu.SemaphoreType.DMA,
        pltpu.SemaphoreType.DMA,
    ],
)
def _emb_lookup(idx_hbm, table_hbm, out_hbm, idx_v, rows_v, gsem, osem):
    wid = lax.axis_index("s") * NC + lax.axis_index("c")
    base = wid * PER_W

    pltpu.sync_copy(idx_hbm.at[wid], idx_v)

    def start_gather(j, slot):
        pltpu.async_copy(table_hbm.at[idx_v.at[j]], rows_v.at[slot], gsem)

    def wait_gather(j, slot):
        pltpu.make_async_copy(
            table_hbm.at[idx_v.at[j]], rows_v.at[slot], gsem
        ).wait()

    def start_store(j, slot):
        pltpu.async_copy(
            rows_v.at[slot], out_hbm.at[pl.ds(base + j * CHUNK, CHUNK)], osem
        )

    def wait_store(j, slot):
        pltpu.make_async_copy(
            rows_v.at[slot], out_hbm.at[pl.ds(base + j * CHUNK, CHUNK)], osem
        ).wait()

    for b in range(NBUF):
        start_gather(b, b)
        if b >= DEPTH:
            wait_gather(b - DEPTH, b - DEPTH)
            start_store(b - DEPTH, b - DEPTH)

    def round_body(g, carry):
        for b in range(NBUF):
            j = g * NBUF + b
            wait_store(j - NBUF, b)
            start_gather(j, b)
            jd = j - DEPTH
            sd = (b - DEPTH) % NBUF
            wait_gather(jd, sd)
            start_store(jd, sd)
        return carry

    lax.fori_loop(1, NG, round_body, 0)

    for b in range(NBUF):
        j = NG * NBUF + b
        wait_store(j - NBUF, b)
        if b < DEPTH:
            jd = j - DEPTH
            sd = (b - DEPTH) % NBUF
            wait_gather(jd, sd)
            start_store(jd, sd)


def kernel(input_, weight):
    idx = input_.astype(jnp.int32).reshape(NW, NCHUNK, CHUNK)
    wpad = jax.lax.pad(
        weight, jnp.float32(0), ((0, 0, 0), (0, PADW - DIM, 0))
    )
    out = _emb_lookup(idx, wpad)
    return out[:, :DIM].reshape(input_.shape[0], input_.shape[1], DIM)


# speedup vs baseline: 2.9275x; 2.7856x over previous
import functools

import jax
import jax.numpy as jnp
from jax import lax
from jax.experimental import pallas as pl
from jax.experimental.pallas import tpu as pltpu
from jax.experimental.pallas import tpu_sc as plsc

NUM_EMBEDDINGS = 1000000
DIM = 32
TOT = 16384 * 100
NC = 2
NS = 16
NW = NC * NS
PER_W = TOT // NW
CHUNK = 128
NCHUNK = PER_W // CHUNK
NBUF = 4
DEPTH = 2
PADW = 128
NG = NCHUNK // NBUF

_mesh = plsc.VectorSubcoreMesh(core_axis_name="c", subcore_axis_name="s")


@functools.partial(
    pl.kernel,
    mesh=_mesh,
    out_type=jax.ShapeDtypeStruct((TOT, PADW), jnp.float32),
    compiler_params=pltpu.CompilerParams(use_tc_tiling_on_sc=True),
    scratch_types=[
        pltpu.VMEM((NCHUNK, CHUNK), jnp.int32),
        pltpu.VMEM((NBUF, CHUNK, PADW), jnp.float32),
        pltpu.SemaphoreType.DMA,
        pltpu.SemaphoreType.DMA,
    ],
)
def _emb_lookup(idx_hbm, table_hbm, out_hbm, idx_v, rows_v, gsem, osem):
    wid = lax.axis_index("s") * NC + lax.axis_index("c")
    base = wid * PER_W

    pltpu.sync_copy(idx_hbm.at[wid], idx_v)

    def start_gather(j, slot):
        pltpu.async_copy(table_hbm.at[idx_v.at[j]], rows_v.at[slot], gsem)

    def wait_gather(j, slot):
        pltpu.make_async_copy(
            table_hbm.at[idx_v.at[j]], rows_v.at[slot], gsem
        ).wait()

    def start_store(j, slot):
        pltpu.async_copy(
            rows_v.at[slot], out_hbm.at[pl.ds(base + j * CHUNK, CHUNK)], osem
        )

    def wait_store(j, slot):
        pltpu.make_async_copy(
            rows_v.at[slot], out_hbm.at[pl.ds(base + j * CHUNK, CHUNK)], osem
        ).wait()

    for b in range(NBUF):
        start_gather(b, b)
        if b >= DEPTH:
            wait_gather(b - DEPTH, b - DEPTH)
            start_store(b - DEPTH, b - DEPTH)

    def round_body(g, carry):
        for b in range(NBUF):
            j = g * NBUF + b
            wait_store(j - NBUF, b)
            start_gather(j, b)
            jd = j - DEPTH
            sd = (b - DEPTH) % NBUF
            wait_gather(jd, sd)
            start_store(jd, sd)
        return carry

    lax.fori_loop(1, NG, round_body, 0)

    for b in range(NBUF):
        j = NG * NBUF + b
        wait_store(j - NBUF, b)
        if b < DEPTH:
            jd = j - DEPTH
            sd = (b - DEPTH) % NBUF
            wait_gather(jd, sd)
            start_store(jd, sd)


BB = 64
BATCH = 16384
FIELDS = 100


def _compact_body(in_ref, out_ref):
    for bi in range(BB):
        out_ref[bi] = in_ref[pl.ds(bi * FIELDS, FIELDS), :DIM]


def _compact(out128):
    return pl.pallas_call(
        _compact_body,
        grid=(BATCH // BB,),
        in_specs=[pl.BlockSpec((BB * FIELDS, PADW), lambda i: (i, 0))],
        out_specs=pl.BlockSpec((BB, FIELDS, DIM), lambda i: (i, 0, 0)),
        out_shape=jax.ShapeDtypeStruct((BATCH, FIELDS, DIM), jnp.float32),
    )(out128)


def kernel(input_, weight):
    idx = input_.astype(jnp.int32).reshape(NW, NCHUNK, CHUNK)
    wpad = jax.lax.pad(
        weight, jnp.float32(0), ((0, 0, 0), (0, PADW - DIM, 0))
    )
    out128 = _emb_lookup(idx, wpad)
    return _compact(out128)


# speedup vs baseline: 3.9328x; 1.3434x over previous
import functools

import jax
import jax.numpy as jnp
from jax import lax
from jax.experimental import pallas as pl
from jax.experimental.pallas import tpu as pltpu
from jax.experimental.pallas import tpu_sc as plsc

NUM_EMBEDDINGS = 1000000
DIM = 32
PADW = 128
BATCH = 16384
FIELDS = 100
NC = 2
NS = 16
NW = NC * NS
B_PER_W = BATCH // NW
IDXBLK = 64
NIDX = B_PER_W // IDXBLK
NBUF = 4

_mesh = plsc.VectorSubcoreMesh(core_axis_name="c", subcore_axis_name="s")


@functools.partial(
    pl.kernel,
    mesh=_mesh,
    out_type=jax.ShapeDtypeStruct((BATCH, FIELDS, DIM), jnp.float32),
    compiler_params=pltpu.CompilerParams(use_tc_tiling_on_sc=True),
    scratch_types=[
        pltpu.VMEM((2, IDXBLK, FIELDS), jnp.int32),
        pltpu.VMEM((NBUF, FIELDS, PADW), jnp.float32),
        pltpu.VMEM((NBUF, FIELDS, DIM), jnp.float32),
        pltpu.SemaphoreType.DMA,
        pltpu.SemaphoreType.DMA,
        pltpu.SemaphoreType.DMA,
    ],
)
def _emb_lookup(idx_hbm, table_hbm, out_hbm, idx_v, rows_v, cpt_v, isem,
                gsem, osem):
    wid = lax.axis_index("s") * NC + lax.axis_index("c")
    base = wid * B_PER_W

    def start_idx(g, slot):
        pltpu.async_copy(
            idx_hbm.at[wid, pl.ds(g * IDXBLK, IDXBLK)], idx_v.at[slot], isem
        )

    def wait_idx(g, slot):
        pltpu.make_async_copy(
            idx_hbm.at[wid, pl.ds(g * IDXBLK, IDXBLK)], idx_v.at[slot], isem
        ).wait()

    def start_gather(g_slot, bb, slot):
        pltpu.async_copy(
            table_hbm.at[idx_v.at[g_slot, bb]], rows_v.at[slot], gsem
        )

    def wait_gather(g_slot, bb, slot):
        pltpu.make_async_copy(
            table_hbm.at[idx_v.at[g_slot, bb]], rows_v.at[slot], gsem
        ).wait()

    def compact(slot):
        def row(i, carry):
            cpt_v[slot, i, pl.ds(0, 16)] = rows_v[slot, i, pl.ds(0, 16)]
            cpt_v[slot, i, pl.ds(16, 16)] = rows_v[slot, i, pl.ds(16, 16)]
            return carry

        lax.fori_loop(0, FIELDS, row, 0)

    def start_store(b, slot):
        pltpu.async_copy(cpt_v.at[slot], out_hbm.at[b], osem)

    def wait_store(b, slot):
        pltpu.make_async_copy(cpt_v.at[slot], out_hbm.at[b], osem).wait()

    start_idx(0, 0)

    def idx_block(g, carry):
        g_slot = g % 2
        wait_idx(g, g_slot)

        @pl.when(g + 1 < NIDX)
        def _():
            start_idx(g + 1, (g + 1) % 2)

        def slot_free(bb):
            bprev = g * IDXBLK + bb - NBUF

            @pl.when(bprev >= 0)
            def _():
                wait_store(base + bprev, bprev % NBUF)

        for p in range(2):
            slot_free(p)
            start_gather(g_slot, p, p % NBUF)

        def batch(bb, carry):
            b = base + g * IDXBLK + bb
            slot = bb % NBUF

            @pl.when(bb + 2 < IDXBLK)
            def _():
                slot_free(bb + 2)
                start_gather(g_slot, bb + 2, (bb + 2) % NBUF)

            wait_gather(g_slot, bb, slot)
            compact(slot)
            start_store(b, slot)
            return carry

        lax.fori_loop(0, IDXBLK, batch, 0)
        return carry

    lax.fori_loop(0, NIDX, idx_block, 0, unroll=2)

    def drain(k, carry):
        b = base + B_PER_W - NBUF + k
        wait_store(b, (B_PER_W - NBUF + k) % NBUF)
        return carry

    lax.fori_loop(0, NBUF, drain, 0)


def kernel(input_, weight):
    idx = input_.astype(jnp.int32).reshape(NW, B_PER_W, FIELDS)
    wpad = jax.lax.pad(
        weight, jnp.float32(0), ((0, 0, 0), (0, PADW - DIM, 0))
    )
    return _emb_lookup(idx, wpad)


# speedup vs baseline: 3.9445x; 1.0030x over previous
import functools

import jax
import jax.numpy as jnp
from jax import lax
from jax.experimental import pallas as pl
from jax.experimental.pallas import tpu as pltpu
from jax.experimental.pallas import tpu_sc as plsc

NUM_EMBEDDINGS = 1000000
DIM = 32
PADW = 128
BATCH = 16384
FIELDS = 100
NC = 2
NS = 16
NW = NC * NS
B_PER_W = BATCH // NW
IDXBLK = 64
NIDX = B_PER_W // IDXBLK
NBUF = 4

_mesh = plsc.VectorSubcoreMesh(core_axis_name="c", subcore_axis_name="s")


@functools.partial(
    pl.kernel,
    mesh=_mesh,
    out_type=jax.ShapeDtypeStruct((BATCH, FIELDS, DIM), jnp.float32),
    compiler_params=pltpu.CompilerParams(use_tc_tiling_on_sc=True),
    scratch_types=[
        pltpu.VMEM((2, IDXBLK, FIELDS), jnp.int32),
        pltpu.VMEM((NBUF, FIELDS, PADW), jnp.float32),
        pltpu.VMEM((NBUF, FIELDS, DIM), jnp.float32),
        pltpu.SemaphoreType.DMA,
        pltpu.SemaphoreType.DMA,
        pltpu.SemaphoreType.DMA,
    ],
)
def _emb_lookup(idx_hbm, table_hbm, out_hbm, idx_v, rows_v, cpt_v, isem,
                gsem, osem):
    wid = lax.axis_index("s") * NC + lax.axis_index("c")
    base = wid * B_PER_W

    def start_idx(g, slot):
        pltpu.async_copy(
            idx_hbm.at[pl.ds(base + g * IDXBLK, IDXBLK)], idx_v.at[slot], isem
        )

    def wait_idx(g, slot):
        pltpu.make_async_copy(
            idx_hbm.at[pl.ds(base + g * IDXBLK, IDXBLK)], idx_v.at[slot], isem
        ).wait()

    def start_gather(g_slot, bb, slot):
        pltpu.async_copy(
            table_hbm.at[idx_v.at[g_slot, bb]], rows_v.at[slot], gsem
        )

    def wait_gather(g_slot, bb, slot):
        pltpu.make_async_copy(
            table_hbm.at[idx_v.at[g_slot, bb]], rows_v.at[slot], gsem
        ).wait()

    def compact(slot):
        def row(i, carry):
            cpt_v[slot, i, pl.ds(0, 16)] = rows_v[slot, i, pl.ds(0, 16)]
            cpt_v[slot, i, pl.ds(16, 16)] = rows_v[slot, i, pl.ds(16, 16)]
            return carry

        lax.fori_loop(0, FIELDS, row, 0)

    def start_store(b, slot):
        pltpu.async_copy(cpt_v.at[slot], out_hbm.at[b], osem)

    def wait_store(b, slot):
        pltpu.make_async_copy(cpt_v.at[slot], out_hbm.at[b], osem).wait()

    start_idx(0, 0)

    def idx_block(g, carry):
        g_slot = g % 2
        wait_idx(g, g_slot)

        @pl.when(g + 1 < NIDX)
        def _():
            start_idx(g + 1, (g + 1) % 2)

        def slot_free(bb):
            bprev = g * IDXBLK + bb - NBUF

            @pl.when(bprev >= 0)
            def _():
                wait_store(base + bprev, bprev % NBUF)

        for p in range(2):
            slot_free(p)
            start_gather(g_slot, p, p % NBUF)

        def batch(bb, carry):
            b = base + g * IDXBLK + bb
            slot = bb % NBUF

            @pl.when(bb + 2 < IDXBLK)
            def _():
                slot_free(bb + 2)
                start_gather(g_slot, bb + 2, (bb + 2) % NBUF)

            wait_gather(g_slot, bb, slot)
            compact(slot)
            start_store(b, slot)
            return carry

        lax.fori_loop(0, IDXBLK, batch, 0)
        return carry

    lax.fori_loop(0, NIDX, idx_block, 0, unroll=2)

    def drain(k, carry):
        b = base + B_PER_W - NBUF + k
        wait_store(b, (B_PER_W - NBUF + k) % NBUF)
        return carry

    lax.fori_loop(0, NBUF, drain, 0)


def kernel(input_, weight):
    wpad = jax.lax.pad(
        weight, jnp.float32(0), ((0, 0, 0), (0, PADW - DIM, 0))
    )
    return _emb_lookup(input_.astype(jnp.int32), wpad)
